# Initial kernel scaffold; baseline (speedup 1.0000x reference)
import jax
import jax.numpy as jnp
from jax.experimental import pallas as pl


def kernel(x, edge_index, W1, b1, W2, b2):
    raise NotImplementedError("write your pallas kernel here")


# speedup vs baseline: 12.3908x; 12.3908x over previous
import functools

import jax
import jax.numpy as jnp
from jax import lax
from jax.experimental import pallas as pl
from jax.experimental.pallas import tpu as pltpu
from jax.experimental.pallas import tpu_sc as plsc

N = 10000
E = 320000
D1 = 128
D2 = 40
D2P = 128

NC = 2
NS = 16
NW = NC * NS
EPW = E // NW
CHUNK = 80
NCHUNK = EPW // CHUNK
RPT = N // NS


def _make_seg_kernel(d, ones_mode):
    mesh = plsc.VectorSubcoreMesh(core_axis_name="c", subcore_axis_name="s")

    @functools.partial(
        pl.kernel,
        out_type=jax.ShapeDtypeStruct((NC * N, d), jnp.float32),
        mesh=mesh,
        scratch_types=[
            pltpu.VMEM((CHUNK,), jnp.int32),
            pltpu.VMEM((CHUNK,), jnp.int32),
            pltpu.VMEM((CHUNK, d), jnp.float32),
            pltpu.VMEM_SHARED((N, d), jnp.float32),
            pltpu.SemaphoreType.DMA,
        ],
    )
    def seg(src_hbm, dst_hbm, xs_hbm, z_hbm, out_hbm,
            src_v, dst_v, rows_v, acc, sem):
        c = lax.axis_index("c")
        s = lax.axis_index("s")

        pltpu.sync_copy(z_hbm, acc.at[pl.ds(s * RPT, RPT)])

        if ones_mode:
            one = jnp.ones((16,), jnp.float32)
            for r in range(CHUNK):
                for j in range(d // 16):
                    rows_v[r, pl.ds(j * 16, 16)] = one

        plsc.subcore_barrier()

        ebase = (c * NS + s) * EPW

        def body(i, carry):
            base = ebase + i * CHUNK
            pltpu.sync_copy(dst_hbm.at[pl.ds(base, CHUNK)], dst_v)
            if not ones_mode:
                pltpu.sync_copy(src_hbm.at[pl.ds(base, CHUNK)], src_v)
                pltpu.async_copy(xs_hbm.at[src_v], rows_v, sem).wait()
            pltpu.sync_copy(rows_v, acc.at[dst_v], add=True)
            return carry

        lax.fori_loop(0, NCHUNK, body, 0)

        plsc.subcore_barrier()

        @pl.when(s < NS - 1)
        def _():
            pltpu.sync_copy(acc.at[pl.ds(s * 632, 632)],
                            out_hbm.at[pl.ds(c * N + s * 632, 632)])

        @pl.when(s == NS - 1)
        def _():
            pltpu.sync_copy(acc.at[pl.ds(15 * 632, N - 15 * 632)],
                            out_hbm.at[pl.ds(c * N + 15 * 632, N - 15 * 632)])

    return seg


_seg128_ones = _make_seg_kernel(D1, True)
_seg128 = _make_seg_kernel(D1, False)


_BLK = 1000
_GRID = N // _BLK


def _tc_b_kernel(p0_ref, p1_ref, x_ref, w1_ref, xs1_ref, dinv_ref):
    deg = p0_ref[:, :1] + p1_ref[:, :1] + 1.0
    dinv = lax.rsqrt(deg)
    y = jnp.dot(x_ref[...], w1_ref[...], preferred_element_type=jnp.float32)
    xs1_ref[...] = y * dinv
    dinv_ref[...] = dinv


def _tc_d_kernel(p0_ref, p1_ref, xs1_ref, dinv_ref, b1_ref, w2_ref, xs2_ref):
    dinv = dinv_ref[...]
    h = jnp.maximum(dinv * (p0_ref[...] + p1_ref[...] + xs1_ref[...])
                    + b1_ref[...], 0.0)
    xs2_ref[...] = jnp.dot(h, w2_ref[...],
                           preferred_element_type=jnp.float32) * dinv


def _tc_f_kernel(q0_ref, q1_ref, xs2_ref, dinv_ref, b2_ref, out_ref):
    z = dinv_ref[...] * (q0_ref[...] + q1_ref[...] + xs2_ref[...]) + b2_ref[...]
    col = lax.broadcasted_iota(jnp.int32, (_BLK, D2P), 1)
    valid = col < D2
    neg = jnp.float32(-1e30)
    zm = jnp.where(valid, z, neg)
    m = jnp.max(zm, axis=1, keepdims=True)
    ez = jnp.where(valid, jnp.exp(z - m), 0.0)
    ssum = jnp.sum(ez, axis=1, keepdims=True)
    out_ref[...] = z - m - jnp.log(ssum)


def kernel(x, edge_index, W1, b1, W2, b2):
    ei = edge_index.astype(jnp.int32)
    src = ei[0]
    dst = ei[1]

    z128 = jnp.zeros((RPT, D1), jnp.float32)

    degp = _seg128_ones(src, dst, z128, z128)

    xs1, dinv = pl.pallas_call(
        _tc_b_kernel,
        grid=(_GRID,),
        in_specs=[
            pl.BlockSpec((_BLK, D1), lambda i: (i, 0)),
            pl.BlockSpec((_BLK, D1), lambda i: (i + _GRID, 0)),
            pl.BlockSpec((_BLK, D1), lambda i: (i, 0)),
            pl.BlockSpec((D1, D1), lambda i: (0, 0)),
        ],
        out_specs=[
            pl.BlockSpec((_BLK, D1), lambda i: (i, 0)),
            pl.BlockSpec((_BLK, 1), lambda i: (i, 0)),
        ],
        out_shape=[
            jax.ShapeDtypeStruct((N, D1), jnp.float32),
            jax.ShapeDtypeStruct((N, 1), jnp.float32),
        ],
    )(degp, degp, x, W1)

    s1p = _seg128(src, dst, xs1, z128)

    W2p = jnp.concatenate(
        [W2, jnp.zeros((D1, D2P - D2), jnp.float32)], axis=1)
    b1r = b1.reshape(1, D1)
    xs2 = pl.pallas_call(
        _tc_d_kernel,
        grid=(_GRID,),
        in_specs=[
            pl.BlockSpec((_BLK, D1), lambda i: (i, 0)),
            pl.BlockSpec((_BLK, D1), lambda i: (i + _GRID, 0)),
            pl.BlockSpec((_BLK, D1), lambda i: (i, 0)),
            pl.BlockSpec((_BLK, 1), lambda i: (i, 0)),
            pl.BlockSpec((1, D1), lambda i: (0, 0)),
            pl.BlockSpec((D1, D2P), lambda i: (0, 0)),
        ],
        out_specs=pl.BlockSpec((_BLK, D2P), lambda i: (i, 0)),
        out_shape=jax.ShapeDtypeStruct((N, D2P), jnp.float32),
    )(s1p, s1p, xs1, dinv, b1r, W2p)

    s2p = _seg128(src, dst, xs2, z128)

    b2r = jnp.concatenate(
        [b2, jnp.zeros((D2P - D2,), jnp.float32)]).reshape(1, D2P)
    outp = pl.pallas_call(
        _tc_f_kernel,
        grid=(_GRID,),
        in_specs=[
            pl.BlockSpec((_BLK, D2P), lambda i: (i, 0)),
            pl.BlockSpec((_BLK, D2P), lambda i: (i + _GRID, 0)),
            pl.BlockSpec((_BLK, D2P), lambda i: (i, 0)),
            pl.BlockSpec((_BLK, 1), lambda i: (i, 0)),
            pl.BlockSpec((1, D2P), lambda i: (0, 0)),
        ],
        out_specs=pl.BlockSpec((_BLK, D2P), lambda i: (i, 0)),
        out_shape=jax.ShapeDtypeStruct((N, D2P), jnp.float32),
    )(s2p, s2p, xs2, dinv, b2r)

    return outp[:, :D2]


# speedup vs baseline: 30.0069x; 2.4217x over previous
import functools

import jax
import jax.numpy as jnp
from jax import lax
from jax.experimental import pallas as pl
from jax.experimental.pallas import tpu as pltpu
from jax.experimental.pallas import tpu_sc as plsc

N = 10000
E = 320000
D1 = 128
D2 = 40
D2P = 128

NC = 2
NS = 16
NW = NC * NS
EPW = E // NW
CHUNK = 80
NCHUNK = EPW // CHUNK
RPT = N // NS


NB = 4


def _make_seg_kernel(d, ones_mode):
    mesh = plsc.VectorSubcoreMesh(core_axis_name="c", subcore_axis_name="s")

    n_rows_buf = 1 if ones_mode else NB
    scratch = [
        [pltpu.VMEM((CHUNK,), jnp.int32)] * NB,
        [pltpu.VMEM((CHUNK,), jnp.int32)] * NB,
        [pltpu.VMEM((CHUNK, d), jnp.float32)] * n_rows_buf,
        pltpu.VMEM_SHARED((N, d), jnp.float32),
        [pltpu.SemaphoreType.DMA] * NB,
        [pltpu.SemaphoreType.DMA] * NB,
        [pltpu.SemaphoreType.DMA] * NB,
    ]

    @functools.partial(
        pl.kernel,
        out_type=jax.ShapeDtypeStruct((NC * N, d), jnp.float32),
        mesh=mesh,
        scratch_types=scratch,
    )
    def seg(src_hbm, dst_hbm, xs_hbm, z_hbm, out_hbm,
            src_v, dst_v, rows_v, acc, sem_i, sem_g, sem_s):
        c = lax.axis_index("c")
        s = lax.axis_index("s")
        w = c * NS + s
        ebase = w * EPW

        pltpu.sync_copy(z_hbm, acc.at[pl.ds(s * RPT, RPT)])

        if ones_mode:
            one = jnp.ones((16,), jnp.float32)

            def fill(i, carry):
                r = i // (d // 16)
                col = (i % (d // 16)) * 16
                rows_v[0][r, pl.ds(col, 16)] = one
                return carry

            lax.fori_loop(0, CHUNK * (d // 16), fill, 0)

        plsc.subcore_barrier()

        def idx_start(u, k):
            base = ebase + u * CHUNK
            pltpu.async_copy(dst_hbm.at[pl.ds(base, CHUNK)], dst_v[k], sem_i[k])
            if not ones_mode:
                pltpu.async_copy(src_hbm.at[pl.ds(base, CHUNK)], src_v[k],
                                 sem_i[k])

        def idx_wait(k):
            pltpu.make_async_copy(dst_hbm.at[pl.ds(0, CHUNK)], dst_v[k],
                                  sem_i[k]).wait()
            if not ones_mode:
                pltpu.make_async_copy(src_hbm.at[pl.ds(0, CHUNK)], src_v[k],
                                      sem_i[k]).wait()

        def gather_start(k):
            pltpu.async_copy(xs_hbm.at[src_v[k]], rows_v[k], sem_g[k])

        def gather_wait(k):
            pltpu.make_async_copy(xs_hbm.at[src_v[k]], rows_v[k],
                                  sem_g[k]).wait()

        def scatter_start(k):
            rows = rows_v[0] if ones_mode else rows_v[k]
            pltpu.async_copy(rows, acc.at[dst_v[k]], sem_s[k], add=True)

        def scatter_wait(k):
            rows = rows_v[0] if ones_mode else rows_v[k]
            pltpu.make_async_copy(rows, acc.at[dst_v[k]], sem_s[k]).wait()

        nslots = NCHUNK + 3
        assert nslots % NB == 0

        def body(t, carry):
            for k in range(NB):
                u = t * NB + k

                @pl.when(u >= NB)
                def _():
                    scatter_wait(k)

                @pl.when(u < NCHUNK)
                def _():
                    idx_start(u, k)

                kk = (k + NB - 1) % NB

                @pl.when((u >= 1) & (u <= NCHUNK))
                def _():
                    idx_wait(kk)
                    if ones_mode:
                        scatter_start(kk)
                    else:
                        gather_start(kk)

                if not ones_mode:
                    km = (k + NB - 3) % NB

                    @pl.when(u >= 3)
                    def _():
                        gather_wait(km)
                        scatter_start(km)
            return carry

        lax.fori_loop(0, nslots // NB, body, 0)
        scatter_wait((NCHUNK - 1) % NB)

        plsc.subcore_barrier()

        @pl.when(s < NS - 1)
        def _():
            pltpu.sync_copy(acc.at[pl.ds(s * 632, 632)],
                            out_hbm.at[pl.ds(c * N + s * 632, 632)])

        @pl.when(s == NS - 1)
        def _():
            pltpu.sync_copy(acc.at[pl.ds(15 * 632, N - 15 * 632)],
                            out_hbm.at[pl.ds(c * N + 15 * 632, N - 15 * 632)])

    return seg


_seg128_ones = _make_seg_kernel(D1, True)
_seg128 = _make_seg_kernel(D1, False)


_BLK = 1000
_GRID = N // _BLK


def _tc_b_kernel(p0_ref, p1_ref, x_ref, w1_ref, xs1_ref, dinv_ref):
    deg = p0_ref[:, :1] + p1_ref[:, :1] + 1.0
    dinv = lax.rsqrt(deg)
    y = jnp.dot(x_ref[...], w1_ref[...], preferred_element_type=jnp.float32)
    xs1_ref[...] = y * dinv
    dinv_ref[...] = dinv


def _tc_d_kernel(p0_ref, p1_ref, xs1_ref, dinv_ref, b1_ref, w2_ref, xs2_ref):
    dinv = dinv_ref[...]
    h = jnp.maximum(dinv * (p0_ref[...] + p1_ref[...] + xs1_ref[...])
                    + b1_ref[...], 0.0)
    xs2_ref[...] = jnp.dot(h, w2_ref[...],
                           preferred_element_type=jnp.float32) * dinv


def _tc_f_kernel(q0_ref, q1_ref, xs2_ref, dinv_ref, b2_ref, out_ref):
    z = dinv_ref[...] * (q0_ref[...] + q1_ref[...] + xs2_ref[...]) + b2_ref[...]
    col = lax.broadcasted_iota(jnp.int32, (_BLK, D2P), 1)
    valid = col < D2
    neg = jnp.float32(-1e30)
    zm = jnp.where(valid, z, neg)
    m = jnp.max(zm, axis=1, keepdims=True)
    ez = jnp.where(valid, jnp.exp(z - m), 0.0)
    ssum = jnp.sum(ez, axis=1, keepdims=True)
    out_ref[...] = z - m - jnp.log(ssum)


def kernel(x, edge_index, W1, b1, W2, b2):
    ei = edge_index.astype(jnp.int32)
    src = ei[0]
    dst = ei[1]

    z128 = jnp.zeros((RPT, D1), jnp.float32)

    degp = _seg128_ones(src, dst, z128, z128)

    xs1, dinv = pl.pallas_call(
        _tc_b_kernel,
        grid=(_GRID,),
        in_specs=[
            pl.BlockSpec((_BLK, D1), lambda i: (i, 0)),
            pl.BlockSpec((_BLK, D1), lambda i: (i + _GRID, 0)),
            pl.BlockSpec((_BLK, D1), lambda i: (i, 0)),
            pl.BlockSpec((D1, D1), lambda i: (0, 0)),
        ],
        out_specs=[
            pl.BlockSpec((_BLK, D1), lambda i: (i, 0)),
            pl.BlockSpec((_BLK, 1), lambda i: (i, 0)),
        ],
        out_shape=[
            jax.ShapeDtypeStruct((N, D1), jnp.float32),
            jax.ShapeDtypeStruct((N, 1), jnp.float32),
        ],
    )(degp, degp, x, W1)

    s1p = _seg128(src, dst, xs1, z128)

    W2p = jnp.concatenate(
        [W2, jnp.zeros((D1, D2P - D2), jnp.float32)], axis=1)
    b1r = b1.reshape(1, D1)
    xs2 = pl.pallas_call(
        _tc_d_kernel,
        grid=(_GRID,),
        in_specs=[
            pl.BlockSpec((_BLK, D1), lambda i: (i, 0)),
            pl.BlockSpec((_BLK, D1), lambda i: (i + _GRID, 0)),
            pl.BlockSpec((_BLK, D1), lambda i: (i, 0)),
            pl.BlockSpec((_BLK, 1), lambda i: (i, 0)),
            pl.BlockSpec((1, D1), lambda i: (0, 0)),
            pl.BlockSpec((D1, D2P), lambda i: (0, 0)),
        ],
        out_specs=pl.BlockSpec((_BLK, D2P), lambda i: (i, 0)),
        out_shape=jax.ShapeDtypeStruct((N, D2P), jnp.float32),
    )(s1p, s1p, xs1, dinv, b1r, W2p)

    s2p = _seg128(src, dst, xs2, z128)

    b2r = jnp.concatenate(
        [b2, jnp.zeros((D2P - D2,), jnp.float32)]).reshape(1, D2P)
    outp = pl.pallas_call(
        _tc_f_kernel,
        grid=(_GRID,),
        in_specs=[
            pl.BlockSpec((_BLK, D2P), lambda i: (i, 0)),
            pl.BlockSpec((_BLK, D2P), lambda i: (i + _GRID, 0)),
            pl.BlockSpec((_BLK, D2P), lambda i: (i, 0)),
            pl.BlockSpec((_BLK, 1), lambda i: (i, 0)),
            pl.BlockSpec((1, D2P), lambda i: (0, 0)),
        ],
        out_specs=pl.BlockSpec((_BLK, D2P), lambda i: (i, 0)),
        out_shape=jax.ShapeDtypeStruct((N, D2P), jnp.float32),
    )(s2p, s2p, xs2, dinv, b2r)

    return outp[:, :D2]


# speedup vs baseline: 39.8858x; 1.3292x over previous
import functools

import jax
import jax.numpy as jnp
from jax import lax
from jax.experimental import pallas as pl
from jax.experimental.pallas import tpu as pltpu
from jax.experimental.pallas import tpu_sc as plsc

N = 10000
E = 320000
D1 = 128
D2 = 40
D2P = 48

NC = 2
NS = 16
NW = NC * NS
EPW = E // NW
CHUNK = 80
NCHUNK = EPW // CHUNK
RPT = N // NS
NB = 4

_UNTILED = pltpu.CompilerParams(use_tc_tiling_on_sc=False)


def _make_seg_kernel(d):
    mesh = plsc.VectorSubcoreMesh(core_axis_name="c", subcore_axis_name="s")

    scratch = [
        [pltpu.VMEM((CHUNK,), jnp.int32)] * NB,
        [pltpu.VMEM((CHUNK,), jnp.int32)] * NB,
        [pltpu.VMEM((CHUNK, d), jnp.float32)] * NB,
        pltpu.VMEM_SHARED((N, d), jnp.float32),
        [pltpu.SemaphoreType.DMA] * NB,
        [pltpu.SemaphoreType.DMA] * NB,
        [pltpu.SemaphoreType.DMA] * NB,
    ]

    @functools.partial(
        pl.kernel,
        out_type=jax.ShapeDtypeStruct((NC * N, d), jnp.float32),
        mesh=mesh,
        scratch_types=scratch,
        compiler_params=_UNTILED,
    )
    def seg(eidx_hbm, xs_hbm, z_hbm, out_hbm,
            src_v, dst_v, rows_v, acc, sem_i, sem_g, sem_s):
        c = lax.axis_index("c")
        s = lax.axis_index("s")
        w = c * NS + s
        ebase = w * EPW

        pltpu.sync_copy(z_hbm, acc.at[pl.ds(s * RPT, RPT)])

        plsc.subcore_barrier()

        def idx_start(u, k):
            base = ebase + u * CHUNK
            pltpu.async_copy(eidx_hbm.at[pl.ds(E + base, CHUNK)], dst_v[k],
                             sem_i[k])
            pltpu.async_copy(eidx_hbm.at[pl.ds(base, CHUNK)], src_v[k],
                             sem_i[k])

        def idx_wait(k):
            pltpu.make_async_copy(eidx_hbm.at[pl.ds(0, CHUNK)], dst_v[k],
                                  sem_i[k]).wait()
            pltpu.make_async_copy(eidx_hbm.at[pl.ds(0, CHUNK)], src_v[k],
                                  sem_i[k]).wait()

        def gather_start(k):
            pltpu.async_copy(xs_hbm.at[src_v[k]], rows_v[k], sem_g[k])

        def gather_wait(k):
            pltpu.make_async_copy(xs_hbm.at[src_v[k]], rows_v[k],
                                  sem_g[k]).wait()

        def scatter_start(k):
            pltpu.async_copy(rows_v[k], acc.at[dst_v[k]], sem_s[k], add=True)

        def scatter_wait(k):
            pltpu.make_async_copy(rows_v[k], acc.at[dst_v[k]], sem_s[k]).wait()

        nslots = NCHUNK + 3
        assert nslots % NB == 0

        def body(t, carry):
            for k in range(NB):
                u = t * NB + k

                @pl.when(u >= NB)
                def _():
                    scatter_wait(k)

                @pl.when(u < NCHUNK)
                def _():
                    idx_start(u, k)

                kk = (k + NB - 1) % NB

                @pl.when((u >= 1) & (u <= NCHUNK))
                def _():
                    idx_wait(kk)
                    gather_start(kk)

                km = (k + NB - 3) % NB

                @pl.when(u >= 3)
                def _():
                    gather_wait(km)
                    scatter_start(km)
            return carry

        lax.fori_loop(0, nslots // NB, body, 0)
        scatter_wait((NCHUNK - 1) % NB)

        plsc.subcore_barrier()

        @pl.when(s < NS - 1)
        def _():
            pltpu.sync_copy(acc.at[pl.ds(s * 632, 632)],
                            out_hbm.at[pl.ds(c * N + s * 632, 632)])

        @pl.when(s == NS - 1)
        def _():
            pltpu.sync_copy(acc.at[pl.ds(15 * 632, N - 15 * 632)],
                            out_hbm.at[pl.ds(c * N + 15 * 632, N - 15 * 632)])

    return seg


_seg128 = _make_seg_kernel(D1)
_seg48 = _make_seg_kernel(D2P)

DROWS = 80
DCHK = 400
NDCHK = EPW // DCHK

_deg_mesh = plsc.VectorSubcoreMesh(core_axis_name="c", subcore_axis_name="s")


@functools.partial(
    pl.kernel,
    out_type=jax.ShapeDtypeStruct((NC * DROWS, D1), jnp.float32),
    mesh=_deg_mesh,
    scratch_types=[
        [pltpu.VMEM((DCHK,), jnp.int32)] * NB,
        pltpu.VMEM((DROWS, D1), jnp.float32),
        pltpu.VMEM((DROWS,), jnp.int32),
        pltpu.VMEM_SHARED((DROWS, D1), jnp.float32),
        [pltpu.SemaphoreType.DMA] * NB,
        pltpu.SemaphoreType.DMA,
    ],
    compiler_params=pltpu.CompilerParams(use_tc_tiling_on_sc=False,
                                         needs_layout_passes=False),
)
def _deg_kernel(eidx_hbm, z_hbm, out_hbm, dst_v, deg_v, rix_v, acc,
                sem_i, sem_r):
    c = lax.axis_index("c")
    s = lax.axis_index("s")
    w = c * NS + s
    ebase = w * EPW

    pltpu.sync_copy(z_hbm.at[pl.ds(0, DROWS)], deg_v)

    @pl.when(s == 0)
    def _():
        pltpu.sync_copy(z_hbm.at[pl.ds(0, DROWS)], acc)

    def mk(i, carry):
        rix_v[pl.ds(i * 16, 16)] = (
            lax.broadcasted_iota(jnp.int32, (16,), 0) + i * 16)
        return carry

    lax.fori_loop(0, DROWS // 16, mk, 0)

    plsc.subcore_barrier()

    one16 = jnp.ones((16,), jnp.float32)

    def idx_start(u, k):
        base = E + ebase + u * DCHK
        pltpu.async_copy(eidx_hbm.at[pl.ds(base, DCHK)], dst_v[k], sem_i[k])

    def idx_wait(k):
        pltpu.make_async_copy(eidx_hbm.at[pl.ds(0, DCHK)], dst_v[k],
                              sem_i[k]).wait()

    def body(t, carry):
        for k in range(NB):
            u = t * NB + k

            @pl.when(u < NDCHK)
            def _():
                idx_start(u, k)

            kk = (k + NB - 1) % NB

            @pl.when((u >= 1) & (u <= NDCHK))
            def _():
                idx_wait(kk)
                for r in range(DCHK // 16):
                    v = dst_v[kk][pl.ds(r * 16, 16)]
                    plsc.addupdate_scatter(
                        deg_v,
                        [lax.shift_right_logical(v, 7), v & 127],
                        one16)
        return carry

    lax.fori_loop(0, (NDCHK + 3) // NB, body, 0)

    pltpu.async_copy(deg_v, acc.at[rix_v], sem_r, add=True).wait()
    plsc.subcore_barrier()

    @pl.when(s < 10)
    def _():
        pltpu.sync_copy(acc.at[pl.ds(s * 8, 8)],
                        out_hbm.at[pl.ds(c * DROWS + s * 8, 8)])


def _tc_b_kernel(p0_ref, p1_ref, x_ref, w1_ref, xs1_ref, dinv_ref):
    deg = p0_ref[...] + p1_ref[...] + 1.0
    dinv = lax.rsqrt(deg)
    y = jnp.dot(x_ref[...], w1_ref[...], preferred_element_type=jnp.float32)
    xs1_ref[...] = y * dinv
    dinv_ref[...] = dinv


def _tc_d_kernel(sp_ref, xs1_ref, dinv_ref, b1_ref, w2_ref, xs2_ref):
    dinv = dinv_ref[...]
    agg = sp_ref[:N, :] + sp_ref[N:, :] + xs1_ref[...]
    h = jnp.maximum(dinv * agg + b1_ref[...], 0.0)
    xs2_ref[...] = jnp.dot(h, w2_ref[...],
                           preferred_element_type=jnp.float32) * dinv


def _tc_f_kernel(sp_ref, xs2_ref, dinv_ref, b2_ref, out_ref):
    z = dinv_ref[...] * (sp_ref[:N, :] + sp_ref[N:, :] + xs2_ref[...]) \
        + b2_ref[...]
    col = lax.broadcasted_iota(jnp.int32, (N, D2P), 1)
    valid = col < D2
    zm = jnp.where(valid, z, jnp.float32(-1e30))
    m = jnp.max(zm, axis=1, keepdims=True)
    ez = jnp.where(valid, jnp.exp(z - m), 0.0)
    ssum = jnp.sum(ez, axis=1, keepdims=True)
    out_ref[...] = (z - m - jnp.log(ssum))[:, :D2]


def kernel(x, edge_index, W1, b1, W2, b2):
    eflat = edge_index.astype(jnp.int32).reshape(2 * E)

    z128 = jnp.zeros((RPT, D1), jnp.float32)
    z48 = jnp.zeros((RPT, D2P), jnp.float32)

    deggrid = _deg_kernel(eflat, z128)
    degflat = deggrid.reshape(NC, DROWS * D1)
    p0 = degflat[0, :N].reshape(N, 1)
    p1 = degflat[1, :N].reshape(N, 1)

    xs1, dinv = pl.pallas_call(
        _tc_b_kernel,
        out_shape=[
            jax.ShapeDtypeStruct((N, D1), jnp.float32),
            jax.ShapeDtypeStruct((N, 1), jnp.float32),
        ],
    )(p0, p1, x, W1)

    s1p = _seg128(eflat, xs1, z128)

    W2p = jnp.concatenate(
        [W2, jnp.zeros((D1, D2P - D2), jnp.float32)], axis=1)
    b1r = b1.reshape(1, D1)
    xs2 = pl.pallas_call(
        _tc_d_kernel,
        out_shape=jax.ShapeDtypeStruct((N, D2P), jnp.float32),
    )(s1p, xs1, dinv, b1r, W2p)

    s2p = _seg48(eflat, xs2, z48)

    b2r = jnp.concatenate(
        [b2, jnp.zeros((D2P - D2,), jnp.float32)]).reshape(1, D2P)
    out = pl.pallas_call(
        _tc_f_kernel,
        out_shape=jax.ShapeDtypeStruct((N, D2), jnp.float32),
    )(s2p, xs2, dinv, b2r)

    return out
